# baseline (speedup 1.0000x reference)
import functools

import jax
import jax.numpy as jnp
from jax import lax
from jax.experimental import pallas as pl
from jax.experimental.pallas import tpu as pltpu
from jax.experimental.pallas import tpu_sc as plsc

NC, NS = 2, 16
NW = NC * NS
D = 128
CH = 128
CHS = 32
NBUF = 4
NSLOT = 10
IG = 16
CNTW = 128

_mesh = plsc.VectorSubcoreMesh(
    core_axis_name="c", subcore_axis_name="s", num_cores=NC, num_subcores=NS)


def _npad(n):
  per_tile = pl.cdiv(n + 1, NS)
  per_tile = (per_tile + 7) // 8 * 8
  return per_tile * NS


def _make_sums_kernel(nchunk, npad):
  rows_per_tile = npad // NS
  ngroups = nchunk // IG

  @functools.partial(
      pl.kernel,
      out_type=jax.ShapeDtypeStruct((NC, npad, D), jnp.float32),
      mesh=_mesh,
      scratch_types=[
          pltpu.VMEM((2, IG, CHS), jnp.int32),
          pltpu.VMEM((2, IG, CHS), jnp.int32),
          [pltpu.VMEM((CHS, D), jnp.float32) for _ in range(NSLOT)],
          pltpu.VMEM_SHARED((npad, D), jnp.float32),
          [pltpu.SemaphoreType.DMA for _ in range(NSLOT)],
          [pltpu.SemaphoreType.DMA for _ in range(NSLOT)],
          [pltpu.SemaphoreType.DMA for _ in range(2)],
          [pltpu.SemaphoreType.DMA for _ in range(2)],
      ],
  )
  def sums(src_hbm, dst_hbm, h_hbm, z_hbm, out_hbm, sidxb, didxb, rows, accum,
           gsems, ssems, isems, jsems):
    cid = lax.axis_index("c")
    sid = lax.axis_index("s")
    tile = cid * NS + sid
    rslice = pl.ds(sid * rows_per_tile, rows_per_tile)

    def fetch_idx(g, d):
      gsl = pl.ds(g * IG, IG)
      pltpu.async_copy(src_hbm.at[tile, gsl], sidxb.at[d], isems[d])
      pltpu.async_copy(dst_hbm.at[tile, gsl], didxb.at[d], jsems[d])

    def wait_idx(g, d):
      gsl = pl.ds(g * IG, IG)
      pltpu.make_async_copy(src_hbm.at[tile, gsl], sidxb.at[d],
                            isems[d]).wait()
      pltpu.make_async_copy(dst_hbm.at[tile, gsl], didxb.at[d],
                            jsems[d]).wait()

    fetch_idx(0, 0)
    pltpu.sync_copy(z_hbm.at[rslice], accum.at[rslice])
    plsc.subcore_barrier()

    def do_group(g, d):
      wait_idx(g, d)

      @pl.when(g + 1 < ngroups)
      def _():
        fetch_idx(g + 1, 1 - d)

      for b in range(NSLOT):
        pltpu.async_copy(h_hbm.at[sidxb.at[d, b]], rows[b], gsems[b])
      for k in range(IG):
        b = k % NSLOT
        pltpu.make_async_copy(h_hbm.at[sidxb.at[d, k]], rows[b],
                              gsems[b]).wait()
        pltpu.async_copy(rows[b], accum.at[didxb.at[d, k]], ssems[b],
                         add=True)
        kp = k - 1
        if kp >= 0 and kp + NSLOT < IG:
          bp = kp % NSLOT
          pltpu.make_async_copy(rows[bp], accum.at[didxb.at[d, kp]],
                                ssems[bp]).wait()
          pltpu.async_copy(h_hbm.at[sidxb.at[d, kp + NSLOT]], rows[bp],
                           gsems[bp])
      for j in range(IG - NSLOT, IG):
        pltpu.make_async_copy(rows[j % NSLOT], accum.at[didxb.at[d, j]],
                              ssems[j % NSLOT]).wait()

    def pair(gp, carry):
      do_group(gp * 2, 0)
      do_group(gp * 2 + 1, 1)
      return carry

    lax.fori_loop(0, ngroups // 2, pair, 0)
    plsc.subcore_barrier()
    pltpu.sync_copy(accum.at[rslice], out_hbm.at[cid, rslice])

  return sums


def _make_counts_kernel(nchunk, npad):
  rows_per_tile = npad // NS

  @functools.partial(
      pl.kernel,
      out_type=jax.ShapeDtypeStruct((NC * npad, CNTW), jnp.float32),
      mesh=_mesh,
      scratch_types=[
          pltpu.VMEM((nchunk, CH), jnp.int32),
          pltpu.VMEM((CH, CNTW), jnp.float32),
          pltpu.VMEM_SHARED((npad, CNTW), jnp.float32),
          [pltpu.SemaphoreType.DMA for _ in range(NBUF)],
      ],
  )
  def counts(dst_hbm, ones_hbm, z_hbm, out_hbm, didx, ones_v, accum, sems):
    cid = lax.axis_index("c")
    sid = lax.axis_index("s")
    tile = cid * NS + sid
    rslice = pl.ds(sid * rows_per_tile, rows_per_tile)
    pltpu.sync_copy(dst_hbm.at[tile], didx)
    pltpu.sync_copy(ones_hbm, ones_v)
    pltpu.sync_copy(z_hbm.at[rslice], accum.at[rslice])
    plsc.subcore_barrier()

    for b in range(NBUF):
      pltpu.async_copy(ones_v, accum.at[didx.at[b]], sems[b], add=True)

    def group(g, carry):
      for b in range(NBUF):
        j = g * NBUF + b
        pltpu.make_async_copy(ones_v, accum.at[didx.at[j]], sems[b]).wait()
        nxt = j + NBUF

        @pl.when(nxt < nchunk)
        def _():
          pltpu.async_copy(ones_v, accum.at[didx.at[nxt]], sems[b], add=True)

      return carry

    lax.fori_loop(0, nchunk // NBUF, group, 0)
    plsc.subcore_barrier()
    out_base = pl.multiple_of(cid * npad + sid * rows_per_tile, 8)
    pltpu.sync_copy(accum.at[rslice],
                    out_hbm.at[pl.ds(out_base, rows_per_tile)])

  return counts


def _dense_body(p_ref, cnt_ref, h_ref, wl_ref, bl_ref, wr_ref, g_ref, be_ref,
                out_ref, *, n, npad):
  p = p_ref[0, :n, :] + p_ref[1, :n, :]
  cnt = cnt_ref[:n, 0:1] + cnt_ref[npad:npad + n, 0:1]
  agg = p / jnp.maximum(cnt, 1.0)
  z = (jnp.dot(agg, wl_ref[...], preferred_element_type=jnp.float32)
       + bl_ref[...][None, :]
       + jnp.dot(h_ref[...], wr_ref[...], preferred_element_type=jnp.float32))
  mu = jnp.mean(z, axis=0, keepdims=True)
  zc = z - mu
  var = jnp.mean(zc * zc, axis=0, keepdims=True)
  y = g_ref[...][None, :] * zc * lax.rsqrt(var + 1e-5) + be_ref[...][None, :]
  out_ref[...] = jnp.where(y > 0.0, y, jnp.exp(jnp.minimum(y, 0.0)) - 1.0)


def _make_dense_kernel(n, npad):
  return pl.pallas_call(
      functools.partial(_dense_body, n=n, npad=npad),
      out_shape=jax.ShapeDtypeStruct((n, D), jnp.float32),
  )


def kernel(x, edge_index, W_l0, b_l0, W_r0, gamma0, beta0, W_l1, b_l1, W_r1,
           gamma1, beta1, W_l2, b_l2, W_r2, gamma2, beta2):
  n = x.shape[0]
  e = edge_index.shape[1]
  npad = _npad(n)
  nchunk_c = pl.cdiv(e, NW * CH * NBUF) * NBUF
  nchunk_s = pl.cdiv(e, NW * CHS * 2 * IG) * 2 * IG

  src = edge_index[0].astype(jnp.int32)
  dst = edge_index[1].astype(jnp.int32)

  def pad_edges(ix, nchunk, width, fill):
    pad = NW * width * nchunk - e
    if pad:
      ix = jnp.concatenate([ix, jnp.full((pad,), fill, jnp.int32)])
    return ix.reshape(NW, nchunk, width)

  src_s = pad_edges(src, nchunk_s, CHS, 0)
  dst_s = pad_edges(dst, nchunk_s, CHS, n)
  dst_c = pad_edges(dst, nchunk_c, CH, n)

  zsum = jnp.zeros((npad, D), jnp.float32)
  zcnt = jnp.zeros((npad, CNTW), jnp.float32)
  ones_rows = jnp.ones((CH, CNTW), jnp.float32)

  sums_k = _make_sums_kernel(nchunk_s, npad)
  counts_k = _make_counts_kernel(nchunk_c, npad)
  dense_k = _make_dense_kernel(n, npad)

  cnt_parts = counts_k(dst_c, ones_rows, zcnt)

  h = x
  for (wl, bl, wr, g, b) in (
      (W_l0, b_l0, W_r0, gamma0, beta0),
      (W_l1, b_l1, W_r1, gamma1, beta1),
      (W_l2, b_l2, W_r2, gamma2, beta2),
  ):
    h_pad = jnp.pad(h, ((0, npad - n), (0, 0)))
    sums = sums_k(src_s, dst_s, h_pad, zsum)
    h = dense_k(sums, cnt_parts, h, wl, bl, wr, g, b)
  return h


# speedup vs baseline: 1.0895x; 1.0895x over previous
import functools

import jax
import jax.numpy as jnp
from jax import lax
from jax.experimental import pallas as pl
from jax.experimental.pallas import tpu as pltpu
from jax.experimental.pallas import tpu_sc as plsc

NC, NS = 2, 16
NW = NC * NS
D = 128
CH = 128
CHS = 64
NBUF = 4
NSLOT = 5
IG = 16
CNTW = 128

_mesh = plsc.VectorSubcoreMesh(
    core_axis_name="c", subcore_axis_name="s", num_cores=NC, num_subcores=NS)


def _npad(n):
  per_tile = pl.cdiv(n + 1, NS)
  per_tile = (per_tile + 7) // 8 * 8
  return per_tile * NS


def _make_sums_kernel(nchunk, npad):
  rows_per_tile = npad // NS
  ngroups = nchunk // IG

  @functools.partial(
      pl.kernel,
      out_type=jax.ShapeDtypeStruct((NC, npad, D), jnp.float32),
      mesh=_mesh,
      scratch_types=[
          pltpu.VMEM((2, IG, CHS), jnp.int32),
          pltpu.VMEM((2, IG, CHS), jnp.int32),
          [pltpu.VMEM((CHS, D), jnp.float32) for _ in range(NSLOT)],
          pltpu.VMEM_SHARED((npad, D), jnp.float32),
          [pltpu.SemaphoreType.DMA for _ in range(NSLOT)],
          [pltpu.SemaphoreType.DMA for _ in range(NSLOT)],
          [pltpu.SemaphoreType.DMA for _ in range(2)],
          [pltpu.SemaphoreType.DMA for _ in range(2)],
      ],
  )
  def sums(src_hbm, dst_hbm, h_hbm, z_hbm, out_hbm, sidxb, didxb, rows, accum,
           gsems, ssems, isems, jsems):
    cid = lax.axis_index("c")
    sid = lax.axis_index("s")
    tile = cid * NS + sid
    rslice = pl.ds(sid * rows_per_tile, rows_per_tile)

    def fetch_idx(g, d):
      gsl = pl.ds(g * IG, IG)
      pltpu.async_copy(src_hbm.at[tile, gsl], sidxb.at[d], isems[d])
      pltpu.async_copy(dst_hbm.at[tile, gsl], didxb.at[d], jsems[d])

    def wait_idx(g, d):
      gsl = pl.ds(g * IG, IG)
      pltpu.make_async_copy(src_hbm.at[tile, gsl], sidxb.at[d],
                            isems[d]).wait()
      pltpu.make_async_copy(dst_hbm.at[tile, gsl], didxb.at[d],
                            jsems[d]).wait()

    fetch_idx(0, 0)
    pltpu.sync_copy(z_hbm.at[rslice], accum.at[rslice])
    plsc.subcore_barrier()

    def do_group(g, d):
      wait_idx(g, d)

      @pl.when(g + 1 < ngroups)
      def _():
        fetch_idx(g + 1, 1 - d)

      for b in range(NSLOT):
        pltpu.async_copy(h_hbm.at[sidxb.at[d, b]], rows[b], gsems[b])
      for k in range(IG):
        b = k % NSLOT
        pltpu.make_async_copy(h_hbm.at[sidxb.at[d, k]], rows[b],
                              gsems[b]).wait()
        pltpu.async_copy(rows[b], accum.at[didxb.at[d, k]], ssems[b],
                         add=True)
        kp = k - 1
        if kp >= 0 and kp + NSLOT < IG:
          bp = kp % NSLOT
          pltpu.make_async_copy(rows[bp], accum.at[didxb.at[d, kp]],
                                ssems[bp]).wait()
          pltpu.async_copy(h_hbm.at[sidxb.at[d, kp + NSLOT]], rows[bp],
                           gsems[bp])
      for j in range(IG - NSLOT, IG):
        pltpu.make_async_copy(rows[j % NSLOT], accum.at[didxb.at[d, j]],
                              ssems[j % NSLOT]).wait()

    def pair(gp, carry):
      do_group(gp * 2, 0)
      do_group(gp * 2 + 1, 1)
      return carry

    lax.fori_loop(0, ngroups // 2, pair, 0)
    plsc.subcore_barrier()
    pltpu.sync_copy(accum.at[rslice], out_hbm.at[cid, rslice])

  return sums


def _make_counts_kernel(nchunk, npad):
  rows_per_tile = npad // NS

  @functools.partial(
      pl.kernel,
      out_type=jax.ShapeDtypeStruct((NC * npad, CNTW), jnp.float32),
      mesh=_mesh,
      scratch_types=[
          pltpu.VMEM((nchunk, CH), jnp.int32),
          pltpu.VMEM((CH, CNTW), jnp.float32),
          pltpu.VMEM_SHARED((npad, CNTW), jnp.float32),
          [pltpu.SemaphoreType.DMA for _ in range(NBUF)],
      ],
  )
  def counts(dst_hbm, ones_hbm, z_hbm, out_hbm, didx, ones_v, accum, sems):
    cid = lax.axis_index("c")
    sid = lax.axis_index("s")
    tile = cid * NS + sid
    rslice = pl.ds(sid * rows_per_tile, rows_per_tile)
    pltpu.sync_copy(dst_hbm.at[tile], didx)
    pltpu.sync_copy(ones_hbm, ones_v)
    pltpu.sync_copy(z_hbm.at[rslice], accum.at[rslice])
    plsc.subcore_barrier()

    for b in range(NBUF):
      pltpu.async_copy(ones_v, accum.at[didx.at[b]], sems[b], add=True)

    def group(g, carry):
      for b in range(NBUF):
        j = g * NBUF + b
        pltpu.make_async_copy(ones_v, accum.at[didx.at[j]], sems[b]).wait()
        nxt = j + NBUF

        @pl.when(nxt < nchunk)
        def _():
          pltpu.async_copy(ones_v, accum.at[didx.at[nxt]], sems[b], add=True)

      return carry

    lax.fori_loop(0, nchunk // NBUF, group, 0)
    plsc.subcore_barrier()
    out_base = pl.multiple_of(cid * npad + sid * rows_per_tile, 8)
    pltpu.sync_copy(accum.at[rslice],
                    out_hbm.at[pl.ds(out_base, rows_per_tile)])

  return counts


def _dense_body(p_ref, cnt_ref, h_ref, wl_ref, bl_ref, wr_ref, g_ref, be_ref,
                out_ref, *, n, npad):
  p = p_ref[0, :n, :] + p_ref[1, :n, :]
  cnt = cnt_ref[:n, 0:1] + cnt_ref[npad:npad + n, 0:1]
  agg = p / jnp.maximum(cnt, 1.0)
  z = (jnp.dot(agg, wl_ref[...], preferred_element_type=jnp.float32)
       + bl_ref[...][None, :]
       + jnp.dot(h_ref[...], wr_ref[...], preferred_element_type=jnp.float32))
  mu = jnp.mean(z, axis=0, keepdims=True)
  zc = z - mu
  var = jnp.mean(zc * zc, axis=0, keepdims=True)
  y = g_ref[...][None, :] * zc * lax.rsqrt(var + 1e-5) + be_ref[...][None, :]
  out_ref[...] = jnp.where(y > 0.0, y, jnp.exp(jnp.minimum(y, 0.0)) - 1.0)


def _make_dense_kernel(n, npad):
  return pl.pallas_call(
      functools.partial(_dense_body, n=n, npad=npad),
      out_shape=jax.ShapeDtypeStruct((n, D), jnp.float32),
  )


def kernel(x, edge_index, W_l0, b_l0, W_r0, gamma0, beta0, W_l1, b_l1, W_r1,
           gamma1, beta1, W_l2, b_l2, W_r2, gamma2, beta2):
  n = x.shape[0]
  e = edge_index.shape[1]
  npad = _npad(n)
  nchunk_c = pl.cdiv(e, NW * CH * NBUF) * NBUF
  nchunk_s = pl.cdiv(e, NW * CHS * 2 * IG) * 2 * IG

  src = edge_index[0].astype(jnp.int32)
  dst = edge_index[1].astype(jnp.int32)

  def pad_edges(ix, nchunk, width, fill):
    pad = NW * width * nchunk - e
    if pad:
      ix = jnp.concatenate([ix, jnp.full((pad,), fill, jnp.int32)])
    return ix.reshape(NW, nchunk, width)

  src_s = pad_edges(src, nchunk_s, CHS, 0)
  dst_s = pad_edges(dst, nchunk_s, CHS, n)
  dst_c = pad_edges(dst, nchunk_c, CH, n)

  zsum = jnp.zeros((npad, D), jnp.float32)
  zcnt = jnp.zeros((npad, CNTW), jnp.float32)
  ones_rows = jnp.ones((CH, CNTW), jnp.float32)

  sums_k = _make_sums_kernel(nchunk_s, npad)
  counts_k = _make_counts_kernel(nchunk_c, npad)
  dense_k = _make_dense_kernel(n, npad)

  cnt_parts = counts_k(dst_c, ones_rows, zcnt)

  h = x
  for (wl, bl, wr, g, b) in (
      (W_l0, b_l0, W_r0, gamma0, beta0),
      (W_l1, b_l1, W_r1, gamma1, beta1),
      (W_l2, b_l2, W_r2, gamma2, beta2),
  ):
    h_pad = jnp.pad(h, ((0, npad - n), (0, 0)))
    sums = sums_k(src_s, dst_s, h_pad, zsum)
    h = dense_k(sums, cnt_parts, h, wl, bl, wr, g, b)
  return h


# speedup vs baseline: 1.1539x; 1.0591x over previous
import functools

import jax
import jax.numpy as jnp
from jax import lax
from jax.experimental import pallas as pl
from jax.experimental.pallas import tpu as pltpu
from jax.experimental.pallas import tpu_sc as plsc

NC, NS = 2, 16
NW = NC * NS
D = 128
CH = 128
CHS = 64
NBUF = 4
NSLOT = 5
IG = 16
CNTW = 128

_mesh = plsc.VectorSubcoreMesh(
    core_axis_name="c", subcore_axis_name="s", num_cores=NC, num_subcores=NS)


def _npad(n):
  per_tile = pl.cdiv(n + 1, NS)
  per_tile = (per_tile + 7) // 8 * 8
  return per_tile * NS


def _make_sums_kernel(nchunk, npad):
  rows_per_tile = npad // NS
  ngroups = nchunk // IG

  @functools.partial(
      pl.kernel,
      out_type=jax.ShapeDtypeStruct((NC, npad, D), jnp.float32),
      mesh=_mesh,
      scratch_types=[
          pltpu.VMEM((2, IG, CHS), jnp.int32),
          pltpu.VMEM((2, IG, CHS), jnp.int32),
          [pltpu.VMEM((CHS, D), jnp.float32) for _ in range(NSLOT)],
          pltpu.VMEM_SHARED((npad, D), jnp.float32),
          [pltpu.SemaphoreType.DMA for _ in range(NSLOT)],
          [pltpu.SemaphoreType.DMA for _ in range(NSLOT)],
          [pltpu.SemaphoreType.DMA for _ in range(2)],
          [pltpu.SemaphoreType.DMA for _ in range(2)],
      ],
  )
  def sums(src_hbm, dst_hbm, h_hbm, z_hbm, out_hbm, sidxb, didxb, rows, accum,
           gsems, ssems, isems, jsems):
    cid = lax.axis_index("c")
    sid = lax.axis_index("s")
    tile = cid * NS + sid
    rslice = pl.ds(sid * rows_per_tile, rows_per_tile)

    def fetch_idx(g, d):
      gsl = pl.ds(g * IG, IG)
      pltpu.async_copy(src_hbm.at[tile, gsl], sidxb.at[d], isems[d])
      pltpu.async_copy(dst_hbm.at[tile, gsl], didxb.at[d], jsems[d])

    def wait_idx(g, d):
      gsl = pl.ds(g * IG, IG)
      pltpu.make_async_copy(src_hbm.at[tile, gsl], sidxb.at[d],
                            isems[d]).wait()
      pltpu.make_async_copy(dst_hbm.at[tile, gsl], didxb.at[d],
                            jsems[d]).wait()

    fetch_idx(0, 0)
    pltpu.sync_copy(z_hbm.at[rslice], accum.at[rslice])
    plsc.subcore_barrier()

    def do_group(g, d):
      wait_idx(g, d)

      @pl.when(g + 1 < ngroups)
      def _():
        fetch_idx(g + 1, 1 - d)

      for b in range(NSLOT):
        pltpu.async_copy(h_hbm.at[sidxb.at[d, b]], rows[b], gsems[b])
      for k in range(IG):
        b = k % NSLOT
        pltpu.make_async_copy(h_hbm.at[sidxb.at[d, k]], rows[b],
                              gsems[b]).wait()
        pltpu.async_copy(rows[b], accum.at[didxb.at[d, k]], ssems[b],
                         add=True)
        kp = k - 1
        if kp >= 0 and kp + NSLOT < IG:
          bp = kp % NSLOT
          pltpu.make_async_copy(rows[bp], accum.at[didxb.at[d, kp]],
                                ssems[bp]).wait()
          pltpu.async_copy(h_hbm.at[sidxb.at[d, kp + NSLOT]], rows[bp],
                           gsems[bp])
      for j in range(IG - NSLOT, IG):
        pltpu.make_async_copy(rows[j % NSLOT], accum.at[didxb.at[d, j]],
                              ssems[j % NSLOT]).wait()

    def pair(gp, carry):
      do_group(gp * 2, 0)
      do_group(gp * 2 + 1, 1)
      return carry

    lax.fori_loop(0, ngroups // 2, pair, 0)
    plsc.subcore_barrier()
    pltpu.sync_copy(accum.at[rslice], out_hbm.at[cid, rslice])

  return sums


def _make_counts_kernel(nchunk, npad):
  rows_per_tile = npad // NS

  @functools.partial(
      pl.kernel,
      out_type=jax.ShapeDtypeStruct((NC * npad, CNTW), jnp.float32),
      mesh=_mesh,
      scratch_types=[
          pltpu.VMEM((nchunk, CH), jnp.int32),
          pltpu.VMEM((CH, CNTW), jnp.float32),
          pltpu.VMEM_SHARED((npad, CNTW), jnp.float32),
          [pltpu.SemaphoreType.DMA for _ in range(NBUF)],
      ],
  )
  def counts(dst_hbm, ones_hbm, z_hbm, out_hbm, didx, ones_v, accum, sems):
    cid = lax.axis_index("c")
    sid = lax.axis_index("s")
    tile = cid * NS + sid
    rslice = pl.ds(sid * rows_per_tile, rows_per_tile)
    pltpu.sync_copy(dst_hbm.at[tile], didx)
    pltpu.sync_copy(ones_hbm, ones_v)
    pltpu.sync_copy(z_hbm.at[rslice], accum.at[rslice])
    plsc.subcore_barrier()

    for b in range(NBUF):
      pltpu.async_copy(ones_v, accum.at[didx.at[b]], sems[b], add=True)

    def group(g, carry):
      for b in range(NBUF):
        j = g * NBUF + b
        pltpu.make_async_copy(ones_v, accum.at[didx.at[j]], sems[b]).wait()
        nxt = j + NBUF

        @pl.when(nxt < nchunk)
        def _():
          pltpu.async_copy(ones_v, accum.at[didx.at[nxt]], sems[b], add=True)

      return carry

    lax.fori_loop(0, nchunk // NBUF, group, 0)
    plsc.subcore_barrier()
    out_base = pl.multiple_of(cid * npad + sid * rows_per_tile, 8)
    pltpu.sync_copy(accum.at[rslice],
                    out_hbm.at[pl.ds(out_base, rows_per_tile)])

  return counts


def _dense_body(p_ref, cnt_ref, h_ref, wl_ref, bl_ref, wr_ref, g_ref, be_ref,
                out_ref, *, n, npad):
  p = p_ref[0, :n, :] + p_ref[1, :n, :]
  cnt = cnt_ref[:n, 0:1] + cnt_ref[npad:npad + n, 0:1]
  agg = p / jnp.maximum(cnt, 1.0)
  z = (jnp.dot(agg, wl_ref[...], preferred_element_type=jnp.float32)
       + bl_ref[...][None, :]
       + jnp.dot(h_ref[...], wr_ref[...], preferred_element_type=jnp.float32))
  mu = jnp.mean(z, axis=0, keepdims=True)
  zc = z - mu
  var = jnp.mean(zc * zc, axis=0, keepdims=True)
  y = g_ref[...][None, :] * zc * lax.rsqrt(var + 1e-5) + be_ref[...][None, :]
  out_ref[...] = jnp.where(y > 0.0, y, jnp.exp(jnp.minimum(y, 0.0)) - 1.0)


def _make_dense_kernel(n, npad):
  return pl.pallas_call(
      functools.partial(_dense_body, n=n, npad=npad),
      out_shape=jax.ShapeDtypeStruct((n, D), jnp.float32),
  )


def kernel(x, edge_index, W_l0, b_l0, W_r0, gamma0, beta0, W_l1, b_l1, W_r1,
           gamma1, beta1, W_l2, b_l2, W_r2, gamma2, beta2):
  n = x.shape[0]
  e = edge_index.shape[1]
  npad = _npad(n)
  nchunk_c = pl.cdiv(e, NW * CH * NBUF) * NBUF
  nchunk_s = pl.cdiv(e, NW * CHS * 2 * IG) * 2 * IG

  src = edge_index[0].astype(jnp.int32)
  dst = edge_index[1].astype(jnp.int32)

  def pad_edges(ix, nchunk, width, fill):
    pad = NW * width * nchunk - e
    if pad:
      ix = jnp.concatenate([ix, jnp.full((pad,), fill, jnp.int32)])
    return ix.reshape(nchunk, NW, width).transpose(1, 0, 2)

  src_s = pad_edges(src, nchunk_s, CHS, 0)
  dst_s = pad_edges(dst, nchunk_s, CHS, n)
  dst_c = pad_edges(dst, nchunk_c, CH, n)

  zsum = jnp.zeros((npad, D), jnp.float32)
  zcnt = jnp.zeros((npad, CNTW), jnp.float32)
  ones_rows = jnp.ones((CH, CNTW), jnp.float32)

  sums_k = _make_sums_kernel(nchunk_s, npad)
  counts_k = _make_counts_kernel(nchunk_c, npad)
  dense_k = _make_dense_kernel(n, npad)

  cnt_parts = counts_k(dst_c, ones_rows, zcnt)

  h = x
  for (wl, bl, wr, g, b) in (
      (W_l0, b_l0, W_r0, gamma0, beta0),
      (W_l1, b_l1, W_r1, gamma1, beta1),
      (W_l2, b_l2, W_r2, gamma2, beta2),
  ):
    h_pad = jnp.pad(h, ((0, npad - n), (0, 0)))
    sums = sums_k(src_s, dst_s, h_pad, zsum)
    h = dense_k(sums, cnt_parts, h, wl, bl, wr, g, b)
  return h
